# baseline (speedup 1.0000x reference)
import functools

import jax
import jax.numpy as jnp
from jax import lax
from jax.experimental import pallas as pl
from jax.experimental.pallas import tpu as pltpu
from jax.experimental.pallas import tpu_sc as plsc

_N = 10000
_D = 128
_L = 5
_E = 320000

_NC = 2
_NS = 16
_NW = _NC * _NS
_EDGE_B = 80
_ITERS = 125
_PER_W = _E // _NW
_ROWS_MAIN = 632
_ROWS_LAST = _N - 15 * _ROWS_MAIN

_mesh = plsc.VectorSubcoreMesh(core_axis_name="c", subcore_axis_name="s")


@functools.partial(
    pl.kernel,
    out_type=jax.ShapeDtypeStruct((_NC, _N, _D), jnp.float32),
    mesh=_mesh,
    scratch_types=[
        pltpu.VMEM((_PER_W,), jnp.int32),
        pltpu.VMEM((_PER_W,), jnp.int32),
        pltpu.VMEM((_EDGE_B, _D), jnp.float32),
        pltpu.VMEM((_EDGE_B, _D), jnp.float32),
        pltpu.VMEM_SHARED((_N, _D), jnp.float32),
        pltpu.SemaphoreType.DMA,
        pltpu.SemaphoreType.DMA,
        pltpu.SemaphoreType.DMA,
        pltpu.SemaphoreType.DMA,
    ],
)
def _sc_aggregate(x_hbm, edge_hbm, zeros_hbm, out_hbm,
                  src_v, dst_v, rows0, rows1, acc_sh,
                  gsem0, gsem1, ssem0, ssem1):
    c = lax.axis_index("c")
    s = lax.axis_index("s")
    wid = s * _NC + c
    row0 = s * _ROWS_MAIN

    @pl.when(s < _NS - 1)
    def _():
        pltpu.sync_copy(zeros_hbm, acc_sh.at[pl.ds(row0, _ROWS_MAIN)])

    @pl.when(s == _NS - 1)
    def _():
        pltpu.sync_copy(zeros_hbm.at[pl.ds(0, _ROWS_LAST)],
                        acc_sh.at[pl.ds(15 * _ROWS_MAIN, _ROWS_LAST)])

    plsc.subcore_barrier()

    pltpu.sync_copy(edge_hbm.at[pl.ds(wid * _PER_W, _PER_W)], src_v)
    pltpu.sync_copy(edge_hbm.at[pl.ds(_E + wid * _PER_W, _PER_W)], dst_v)

    def _gather(j, buf, sem):
        return pltpu.async_copy(
            x_hbm.at[src_v.at[pl.ds(j * _EDGE_B, _EDGE_B)]], buf, sem)

    def _gather_wait(j, buf, sem):
        pltpu.make_async_copy(
            x_hbm.at[src_v.at[pl.ds(j * _EDGE_B, _EDGE_B)]], buf, sem).wait()

    def _scat(j, buf, sem):
        return pltpu.async_copy(
            buf, acc_sh.at[dst_v.at[pl.ds(j * _EDGE_B, _EDGE_B)]], sem,
            add=True)

    def _scat_wait(j, buf, sem):
        pltpu.make_async_copy(
            buf, acc_sh.at[dst_v.at[pl.ds(j * _EDGE_B, _EDGE_B)]],
            sem).wait()

    _gather(0, rows0, gsem0)

    @pl.loop(0, _ITERS // 2)
    def _(jj):
        j0 = jj * 2
        _gather_wait(j0, rows0, gsem0)
        _scat(j0, rows0, ssem0)

        @pl.when(jj > 0)
        def _():
            _scat_wait(j0 - 1, rows1, ssem1)

        _gather(j0 + 1, rows1, gsem1)
        _gather_wait(j0 + 1, rows1, gsem1)
        _scat(j0 + 1, rows1, ssem1)
        _scat_wait(j0, rows0, ssem0)
        _gather(j0 + 2, rows0, gsem0)

    _gather_wait(_ITERS - 1, rows0, gsem0)
    _scat(_ITERS - 1, rows0, ssem0)
    _scat_wait(_ITERS - 2, rows1, ssem1)
    _scat_wait(_ITERS - 1, rows0, ssem0)

    plsc.subcore_barrier()

    @pl.when(s < _NS - 1)
    def _():
        pltpu.sync_copy(acc_sh.at[pl.ds(row0, _ROWS_MAIN)],
                        out_hbm.at[c, pl.ds(row0, _ROWS_MAIN)])

    @pl.when(s == _NS - 1)
    def _():
        pltpu.sync_copy(acc_sh.at[pl.ds(15 * _ROWS_MAIN, _ROWS_LAST)],
                        out_hbm.at[c, pl.ds(15 * _ROWS_MAIN, _ROWS_LAST)])


def _mlp_body(x_ref, p_ref, w_ref, b_ref, g_ref, bt_ref, o_ref):
    h = x_ref[...] + p_ref[0] + p_ref[1]
    for i in range(_L - 1):
        h = jnp.dot(h, w_ref[i], preferred_element_type=jnp.float32,
                    precision=lax.Precision.HIGHEST) + b_ref[i]
        mu = jnp.mean(h, axis=0)
        var = jnp.mean((h - mu) ** 2, axis=0)
        h = g_ref[i] * (h - mu) / jnp.sqrt(var + 1e-5) + bt_ref[i]
        h = jnp.maximum(h, 0.0)
    o_ref[...] = jnp.dot(h, w_ref[_L - 1], preferred_element_type=jnp.float32,
                         precision=lax.Precision.HIGHEST) + b_ref[_L - 1]


def _mlp(x, partial, W, b, gamma, beta):
    return pl.pallas_call(
        _mlp_body,
        out_shape=jax.ShapeDtypeStruct((_N, _D), jnp.float32),
    )(x, partial, W, b, gamma, beta)


def kernel(x, edge_index, W, b, gamma, beta):
    edge_flat = edge_index.astype(jnp.int32).reshape(2 * _E)
    zeros = jnp.zeros((_ROWS_MAIN, _D), jnp.float32)
    partial = _sc_aggregate(x, edge_flat, zeros)
    return _mlp(x, partial, W, b, gamma, beta)


# speedup vs baseline: 1.2979x; 1.2979x over previous
import functools

import jax
import jax.numpy as jnp
from jax import lax
from jax.experimental import pallas as pl
from jax.experimental.pallas import tpu as pltpu
from jax.experimental.pallas import tpu_sc as plsc

_N = 10000
_D = 128
_L = 5
_E = 320000

_NC = 2
_NS = 16
_NW = _NC * _NS
_EDGE_B = 125
_ITERS = 80
_IC = 40
_ROWS_MAIN = 632
_ROWS_LAST = _N - 15 * _ROWS_MAIN

_mesh = plsc.VectorSubcoreMesh(core_axis_name="c", subcore_axis_name="s")


@functools.partial(
    pl.kernel,
    out_type=jax.ShapeDtypeStruct((_NC, _N, _D), jnp.float32),
    mesh=_mesh,
    scratch_types=[
        pltpu.VMEM((_IC, _EDGE_B), jnp.int32),
        pltpu.VMEM((_IC, _EDGE_B), jnp.int32),
        pltpu.VMEM((_EDGE_B, _D), jnp.float32),
        pltpu.VMEM((_EDGE_B, _D), jnp.float32),
        pltpu.VMEM_SHARED((_N, _D), jnp.float32),
        pltpu.SemaphoreType.DMA,
        pltpu.SemaphoreType.DMA,
        pltpu.SemaphoreType.DMA,
        pltpu.SemaphoreType.DMA,
    ],
)
def _sc_aggregate(x_hbm, edge_hbm, zeros_hbm, out_hbm,
                  src_v, dst_v, rows0, rows1, acc_sh,
                  gsem0, gsem1, ssem0, ssem1):
    c = lax.axis_index("c")
    s = lax.axis_index("s")
    wid = s * _NC + c
    row0 = s * _ROWS_MAIN

    @pl.when(s < _NS - 1)
    def _():
        pltpu.sync_copy(zeros_hbm, acc_sh.at[pl.ds(row0, _ROWS_MAIN)])

    @pl.when(s == _NS - 1)
    def _():
        pltpu.sync_copy(zeros_hbm.at[pl.ds(0, _ROWS_LAST)],
                        acc_sh.at[pl.ds(15 * _ROWS_MAIN, _ROWS_LAST)])

    plsc.subcore_barrier()

    @pl.loop(0, _ITERS // _IC)
    def _(ch):
        @pl.when(ch > 0)
        def _():
            pltpu.make_async_copy(rows1, acc_sh.at[dst_v.at[_IC - 1]],
                                  ssem1).wait()

        pltpu.sync_copy(edge_hbm.at[0, wid, pl.ds(ch * _IC, _IC)], src_v)
        pltpu.sync_copy(edge_hbm.at[1, wid, pl.ds(ch * _IC, _IC)], dst_v)
        pltpu.async_copy(x_hbm.at[src_v.at[0]], rows0, gsem0)

        @pl.loop(0, _IC // 2)
        def _(jj):
            j0 = jj * 2
            pltpu.make_async_copy(x_hbm.at[src_v.at[j0]], rows0,
                                  gsem0).wait()
            pltpu.async_copy(rows0, acc_sh.at[dst_v.at[j0]], ssem0,
                             add=True)

            @pl.when(jj > 0)
            def _():
                pltpu.make_async_copy(rows1, acc_sh.at[dst_v.at[j0 - 1]],
                                      ssem1).wait()

            pltpu.async_copy(x_hbm.at[src_v.at[j0 + 1]], rows1, gsem1)
            pltpu.make_async_copy(x_hbm.at[src_v.at[j0 + 1]], rows1,
                                  gsem1).wait()
            pltpu.async_copy(rows1, acc_sh.at[dst_v.at[j0 + 1]], ssem1,
                             add=True)
            pltpu.make_async_copy(rows0, acc_sh.at[dst_v.at[j0]],
                                  ssem0).wait()

            @pl.when(jj < _IC // 2 - 1)
            def _():
                pltpu.async_copy(x_hbm.at[src_v.at[j0 + 2]], rows0, gsem0)

    pltpu.make_async_copy(rows1, acc_sh.at[dst_v.at[_IC - 1]], ssem1).wait()

    plsc.subcore_barrier()

    @pl.when(s < _NS - 1)
    def _():
        pltpu.sync_copy(acc_sh.at[pl.ds(row0, _ROWS_MAIN)],
                        out_hbm.at[c, pl.ds(row0, _ROWS_MAIN)])

    @pl.when(s == _NS - 1)
    def _():
        pltpu.sync_copy(acc_sh.at[pl.ds(15 * _ROWS_MAIN, _ROWS_LAST)],
                        out_hbm.at[c, pl.ds(15 * _ROWS_MAIN, _ROWS_LAST)])


def _mlp_body(x_ref, p_ref, w_ref, b_ref, g_ref, bt_ref, o_ref):
    h = x_ref[...] + p_ref[0] + p_ref[1]
    for i in range(_L - 1):
        h = jnp.dot(h, w_ref[i],
                    preferred_element_type=jnp.float32) + b_ref[i]
        mu = jnp.mean(h, axis=0)
        var = jnp.mean((h - mu) ** 2, axis=0)
        h = g_ref[i] * (h - mu) / jnp.sqrt(var + 1e-5) + bt_ref[i]
        h = jnp.maximum(h, 0.0)
    o_ref[...] = jnp.dot(h, w_ref[_L - 1],
                         preferred_element_type=jnp.float32) + b_ref[_L - 1]


def _mlp(x, partial, W, b, gamma, beta):
    return pl.pallas_call(
        _mlp_body,
        out_shape=jax.ShapeDtypeStruct((_N, _D), jnp.float32),
    )(x, partial, W, b, gamma, beta)


def kernel(x, edge_index, W, b, gamma, beta):
    edge_r = edge_index.astype(jnp.int32).reshape(2, _NW, _ITERS, _EDGE_B)
    zeros = jnp.zeros((_ROWS_MAIN, _D), jnp.float32)
    partial = _sc_aggregate(x, edge_r, zeros)
    return _mlp(x, partial, W, b, gamma, beta)


# speedup vs baseline: 1.6254x; 1.2523x over previous
import functools

import jax
import jax.numpy as jnp
from jax import lax
from jax.experimental import pallas as pl
from jax.experimental.pallas import tpu as pltpu
from jax.experimental.pallas import tpu_sc as plsc

_N = 10000
_D = 128
_L = 5
_E = 320000

_NC = 2
_NS = 16
_NW = _NC * _NS
_EDGE_B = 80
_ITERS = 125
_PER_W = _E // _NW
_ROWS_MAIN = 632
_ROWS_LAST = _N - 15 * _ROWS_MAIN

_mesh = plsc.VectorSubcoreMesh(core_axis_name="c", subcore_axis_name="s")


@functools.partial(
    pl.kernel,
    out_type=jax.ShapeDtypeStruct((_NC, _N, _D), jnp.float32),
    mesh=_mesh,
    scratch_types=[
        pltpu.VMEM((_PER_W,), jnp.int32),
        pltpu.VMEM((_PER_W,), jnp.int32),
        pltpu.VMEM((_EDGE_B, _D), jnp.float32),
        pltpu.VMEM((_EDGE_B, _D), jnp.float32),
        pltpu.VMEM((_EDGE_B, _D), jnp.float32),
        pltpu.VMEM_SHARED((_N, _D), jnp.float32),
        pltpu.SemaphoreType.DMA,
        pltpu.SemaphoreType.DMA,
        pltpu.SemaphoreType.DMA,
        pltpu.SemaphoreType.DMA,
        pltpu.SemaphoreType.DMA,
        pltpu.SemaphoreType.DMA,
    ],
)
def _sc_aggregate(x_hbm, edge_hbm, zeros_hbm, out_hbm,
                  src_v, dst_v, rows0, rows1, rows2, acc_sh,
                  gs0, gs1, gs2, ss0, ss1, ss2):
    c = lax.axis_index("c")
    s = lax.axis_index("s")
    wid = s * _NC + c
    row0 = s * _ROWS_MAIN

    @pl.when(s < _NS - 1)
    def _():
        pltpu.sync_copy(zeros_hbm, acc_sh.at[pl.ds(row0, _ROWS_MAIN)])

    @pl.when(s == _NS - 1)
    def _():
        pltpu.sync_copy(zeros_hbm.at[pl.ds(0, _ROWS_LAST)],
                        acc_sh.at[pl.ds(15 * _ROWS_MAIN, _ROWS_LAST)])

    plsc.subcore_barrier()

    pltpu.sync_copy(edge_hbm.at[pl.ds(wid * _PER_W, _PER_W)], src_v)
    pltpu.sync_copy(edge_hbm.at[pl.ds(_E + wid * _PER_W, _PER_W)], dst_v)

    def _g(j, buf, sem):
        return pltpu.async_copy(
            x_hbm.at[src_v.at[pl.ds(j * _EDGE_B, _EDGE_B)]], buf, sem)

    def _gw(j, buf, sem):
        pltpu.make_async_copy(
            x_hbm.at[src_v.at[pl.ds(j * _EDGE_B, _EDGE_B)]], buf, sem).wait()

    def _s(j, buf, sem):
        return pltpu.async_copy(
            buf, acc_sh.at[dst_v.at[pl.ds(j * _EDGE_B, _EDGE_B)]], sem,
            add=True)

    def _sw(j, buf, sem):
        pltpu.make_async_copy(
            buf, acc_sh.at[dst_v.at[pl.ds(j * _EDGE_B, _EDGE_B)]],
            sem).wait()

    _g(0, rows0, gs0)
    _g(1, rows1, gs1)

    @pl.loop(0, (_ITERS - 2) // 3)
    def _(k):
        j0 = k * 3
        _gw(j0, rows0, gs0)
        _s(j0, rows0, ss0)

        @pl.when(k > 0)
        def _():
            _sw(j0 - 1, rows2, ss2)

        _g(j0 + 2, rows2, gs2)
        _gw(j0 + 1, rows1, gs1)
        _s(j0 + 1, rows1, ss1)
        _sw(j0, rows0, ss0)
        _g(j0 + 3, rows0, gs0)
        _gw(j0 + 2, rows2, gs2)
        _s(j0 + 2, rows2, ss2)
        _sw(j0 + 1, rows1, ss1)
        _g(j0 + 4, rows1, gs1)

    _gw(_ITERS - 2, rows0, gs0)
    _s(_ITERS - 2, rows0, ss0)
    _sw(_ITERS - 3, rows2, ss2)
    _gw(_ITERS - 1, rows1, gs1)
    _s(_ITERS - 1, rows1, ss1)
    _sw(_ITERS - 2, rows0, ss0)
    _sw(_ITERS - 1, rows1, ss1)

    plsc.subcore_barrier()

    @pl.when(s < _NS - 1)
    def _():
        pltpu.sync_copy(acc_sh.at[pl.ds(row0, _ROWS_MAIN)],
                        out_hbm.at[c, pl.ds(row0, _ROWS_MAIN)])

    @pl.when(s == _NS - 1)
    def _():
        pltpu.sync_copy(acc_sh.at[pl.ds(15 * _ROWS_MAIN, _ROWS_LAST)],
                        out_hbm.at[c, pl.ds(15 * _ROWS_MAIN, _ROWS_LAST)])


def _mlp_body(x_ref, p_ref, w_ref, b_ref, g_ref, bt_ref, o_ref):
    h = x_ref[...] + p_ref[0] + p_ref[1]
    for i in range(_L - 1):
        h = jnp.dot(h, w_ref[i],
                    preferred_element_type=jnp.float32) + b_ref[i]
        mu = jnp.mean(h, axis=0)
        var = jnp.mean((h - mu) ** 2, axis=0)
        h = g_ref[i] * (h - mu) / jnp.sqrt(var + 1e-5) + bt_ref[i]
        h = jnp.maximum(h, 0.0)
    o_ref[...] = jnp.dot(h, w_ref[_L - 1],
                         preferred_element_type=jnp.float32) + b_ref[_L - 1]


def _mlp(x, partial, W, b, gamma, beta):
    return pl.pallas_call(
        _mlp_body,
        out_shape=jax.ShapeDtypeStruct((_N, _D), jnp.float32),
    )(x, partial, W, b, gamma, beta)


def kernel(x, edge_index, W, b, gamma, beta):
    edge_flat = edge_index.astype(jnp.int32).reshape(2 * _E)
    zeros = jnp.zeros((_ROWS_MAIN, _D), jnp.float32)
    partial = _sc_aggregate(x, edge_flat, zeros)
    return _mlp(x, partial, W, b, gamma, beta)
